# Initial kernel scaffold; baseline (speedup 1.0000x reference)
import jax
import jax.numpy as jnp
from jax.experimental import pallas as pl


def kernel(input, weight, scale):
    raise NotImplementedError("write your pallas kernel here")


# speedup vs baseline: 1.0129x; 1.0129x over previous
import functools

import jax
import jax.numpy as jnp
from jax import lax
from jax.experimental import pallas as pl
from jax.experimental.pallas import tpu as pltpu
from jax.experimental.pallas import tpu_sc as plsc

NC = 2
NS = 16
NW = NC * NS

C = 1024
IW = 128
G = C // IW


def _sc_embedding(idx2d, weight, scale16, n, d):
    per_w = n // NW
    n_chunks = per_w // C
    grp_per_w = per_w // IW

    mesh = plsc.VectorSubcoreMesh(
        core_axis_name="c", subcore_axis_name="s",
        num_cores=NC, num_subcores=NS)

    @functools.partial(
        pl.kernel,
        out_type=jax.ShapeDtypeStruct((n, d), jnp.float32),
        mesh=mesh,
        compiler_params=pltpu.CompilerParams(use_tc_tiling_on_sc=False),
        scratch_types=[
            pltpu.VMEM((G, IW), jnp.int32),
            pltpu.VMEM((C, 32), jnp.float32),
            pltpu.VMEM((16,), jnp.float32),
            pltpu.SemaphoreType.DMA,
        ],
    )
    def k(idx_hbm, w_hbm, s_hbm, out_hbm, idx_v, rows_v, s_v, sem):
        cid = lax.axis_index("c")
        sid = lax.axis_index("s")
        wid = sid * NC + cid
        pltpu.sync_copy(s_hbm, s_v)
        sf = jnp.exp(s_v[...])
        grp_base = wid * grp_per_w
        row_base = wid * per_w

        def chunk(g, carry):
            pltpu.sync_copy(idx_hbm.at[pl.ds(grp_base + g * G, G)], idx_v)
            copies = [
                pltpu.async_copy(w_hbm.at[idx_v.at[j]],
                                 rows_v.at[pl.ds(j * IW, IW)], sem)
                for j in range(G)
            ]
            for cpy in copies:
                cpy.wait()

            def srow(r, carry2):
                rows_v[r, pl.ds(0, 16)] = rows_v[r, pl.ds(0, 16)] * sf
                rows_v[r, pl.ds(16, 16)] = rows_v[r, pl.ds(16, 16)] * sf
                return carry2

            lax.fori_loop(0, C, srow, 0, unroll=4)
            pltpu.sync_copy(rows_v, out_hbm.at[pl.ds(row_base + g * C, C)])
            return carry

        lax.fori_loop(0, n_chunks, chunk, 0)

    return k(idx2d, weight, scale16)


def kernel(input, weight, scale):
    b, l = input.shape
    d = weight.shape[1]
    n = b * l
    idx2d = input.reshape(n // IW, IW).astype(jnp.int32)
    scale16 = jnp.broadcast_to(scale.astype(jnp.float32), (16,))
    out = _sc_embedding(idx2d, weight, scale16, n, d)
    return out.reshape(b, l, d)


# speedup vs baseline: 1.4275x; 1.4093x over previous
import functools

import jax
import jax.numpy as jnp
from jax import lax
from jax.experimental import pallas as pl
from jax.experimental.pallas import tpu as pltpu
from jax.experimental.pallas import tpu_sc as plsc

NC = 2
NS = 16
NW = NC * NS

B = 16384
L = 50
D = 32
BL = 128
NBH = B // BL
BH_PER_W = NBH // NW


def _sc_embedding(idx_b, weight, scale16):
    mesh = plsc.VectorSubcoreMesh(
        core_axis_name="c", subcore_axis_name="s",
        num_cores=NC, num_subcores=NS)

    n_blocks = BH_PER_W * L

    @functools.partial(
        pl.kernel,
        out_type=jax.ShapeDtypeStruct((B * L * D,), jnp.float32),
        mesh=mesh,
        compiler_params=pltpu.CompilerParams(
            use_tc_tiling_on_sc=False, needs_layout_passes=False),
        scratch_types=[
            pltpu.VMEM((BH_PER_W, L, BL), jnp.int32),
            pltpu.VMEM((BL, D), jnp.float32),
            pltpu.VMEM((4 * 8 * BL,), jnp.float32),
            pltpu.VMEM((16,), jnp.float32),
            pltpu.SemaphoreType.DMA,
            pltpu.SemaphoreType.DMA,
        ],
    )
    def k(idx_hbm, w_hbm, s_hbm, out_hbm, idx_v, rows_v, arr_v, s_v,
          sem_g, sem_w):
        cid = lax.axis_index("c")
        sid = lax.axis_index("s")
        wid = sid * NC + cid
        bh0 = wid * BH_PER_W

        pltpu.sync_copy(s_hbm, s_v)
        sf = jnp.exp(s_v[...])
        pltpu.sync_copy(idx_hbm.at[pl.ds(bh0, BH_PER_W)], idx_v)

        lane = lax.iota(jnp.int32, 16)
        base0 = (lane // 8) * 1024 + (lane % 8) * 128
        base1 = base0 + 2048

        def blk(kk, carry):
            l = kk // BH_PER_W
            bh = kk % BH_PER_W
            pltpu.async_copy(w_hbm.at[idx_v.at[bh, l]], rows_v, sem_g).wait()

            def rearr(bl, c2):
                v0 = rows_v[bl, pl.ds(0, 16)] * sf
                v1 = rows_v[bl, pl.ds(16, 16)] * sf
                plsc.store_scatter(arr_v, [base0 + bl], v0)
                plsc.store_scatter(arr_v, [base1 + bl], v1)
                return c2

            lax.fori_loop(0, BL, rearr, 0, unroll=4)

            ofs = l * (4 * NBH * 1024) + (bh0 + bh) * 1024
            copies = [
                pltpu.async_copy(
                    arr_v.at[pl.ds(ch * 1024, 1024)],
                    out_hbm.at[pl.ds(ofs + ch * (NBH * 1024), 1024)],
                    sem_w)
                for ch in range(4)
            ]
            for cpy in copies:
                cpy.wait()
            return carry

        lax.fori_loop(0, n_blocks, blk, 0)

    return k(idx_b, weight, scale16)


def kernel(input, weight, scale):
    idx_b = input.astype(jnp.int32).reshape(NBH, BL, L).transpose(0, 2, 1)
    scale16 = jnp.broadcast_to(scale.astype(jnp.float32), (16,))
    flat = _sc_embedding(idx_b, weight, scale16)
    out5 = flat.reshape(L, 4, NBH, 8, BL)
    return out5.transpose(2, 4, 0, 1, 3).reshape(B, L, D)


# speedup vs baseline: 1.7025x; 1.1926x over previous
import functools

import jax
import jax.numpy as jnp
from jax import lax
from jax.experimental import pallas as pl
from jax.experimental.pallas import tpu as pltpu
from jax.experimental.pallas import tpu_sc as plsc

NC = 2
NS = 16
NW = NC * NS

B = 16384
L = 50
D = 32
BL = 128
NBH = B // BL
BH_PER_W = NBH // NW
L_STRIDE = 4 * NBH * 1024
CH_STRIDE = NBH * 1024


def _sc_embedding(idx_b, weight, scale16):
    mesh = plsc.VectorSubcoreMesh(
        core_axis_name="c", subcore_axis_name="s",
        num_cores=NC, num_subcores=NS)

    @functools.partial(
        pl.kernel,
        out_type=jax.ShapeDtypeStruct((B * L * D,), jnp.float32),
        mesh=mesh,
        compiler_params=pltpu.CompilerParams(
            use_tc_tiling_on_sc=False, needs_layout_passes=False),
        scratch_types=[
            pltpu.VMEM((BH_PER_W, L, BL), jnp.int32),
            pltpu.VMEM((BH_PER_W, BL, D), jnp.float32),
            pltpu.VMEM((BH_PER_W, 4 * 8 * BL), jnp.float32),
            pltpu.VMEM((16,), jnp.float32),
            pltpu.SemaphoreType.DMA,
            pltpu.SemaphoreType.DMA,
        ],
    )
    def k(idx_hbm, w_hbm, s_hbm, out_hbm, idx_v, rows_v, arr_v, s_v,
          sem_g, sem_w):
        cid = lax.axis_index("c")
        sid = lax.axis_index("s")
        wid = sid * NC + cid
        bh0 = wid * BH_PER_W

        pltpu.sync_copy(s_hbm, s_v)
        sf = jnp.exp(s_v[...])
        pltpu.sync_copy(idx_hbm.at[pl.ds(bh0, BH_PER_W)], idx_v)

        lane = lax.iota(jnp.int32, 16)
        base0 = (lane // 8) * 1024 + (lane % 8) * 128
        base1 = base0 + 2048

        def issue_gather(l, j):
            pltpu.async_copy(w_hbm.at[idx_v.at[j, l]], rows_v.at[j], sem_g)

        def block(l, j, first, last):
            pltpu.make_async_copy(
                w_hbm.at[idx_v.at[j, l]], rows_v.at[j], sem_g).wait()
            if not first:
                for ch in range(4):
                    pltpu.make_async_copy(
                        arr_v.at[j].at[pl.ds(ch * 1024, 1024)],
                        out_hbm.at[pl.ds(ch * 1024, 1024)],
                        sem_w).wait()

            def rearr(bl, c2):
                v0 = rows_v[j, bl, pl.ds(0, 16)] * sf
                v1 = rows_v[j, bl, pl.ds(16, 16)] * sf
                plsc.store_scatter(arr_v.at[j], [base0 + bl], v0)
                plsc.store_scatter(arr_v.at[j], [base1 + bl], v1)
                return c2

            lax.fori_loop(0, BL, rearr, 0, unroll=8)

            ofs = l * L_STRIDE + (bh0 + j) * 1024
            for ch in range(4):
                pltpu.async_copy(
                    arr_v.at[j].at[pl.ds(ch * 1024, 1024)],
                    out_hbm.at[pl.ds(ofs + ch * CH_STRIDE, 1024)],
                    sem_w)
            if not last:
                issue_gather(l + 1, j)

        for j in range(BH_PER_W):
            issue_gather(0, j)
        for j in range(BH_PER_W):
            block(0, j, first=True, last=False)

        def body(l, carry):
            for j in range(BH_PER_W):
                block(l, j, first=False, last=False)
            return carry

        lax.fori_loop(1, L - 1, body, 0)

        for j in range(BH_PER_W):
            block(L - 1, j, first=False, last=True)
        for j in range(BH_PER_W):
            for ch in range(4):
                pltpu.make_async_copy(
                    arr_v.at[j].at[pl.ds(ch * 1024, 1024)],
                    out_hbm.at[pl.ds(ch * 1024, 1024)],
                    sem_w).wait()

    return k(idx_b, weight, scale16)


def kernel(input, weight, scale):
    idx_b = input.astype(jnp.int32).reshape(NBH, BL, L).transpose(0, 2, 1)
    scale16 = jnp.broadcast_to(scale.astype(jnp.float32), (16,))
    flat = _sc_embedding(idx_b, weight, scale16)
    out5 = flat.reshape(L, 4, NBH, 8, BL)
    return out5.transpose(2, 4, 0, 1, 3).reshape(B, L, D)


# speedup vs baseline: 2.6514x; 1.5573x over previous
import functools

import jax
import jax.numpy as jnp
from jax import lax
from jax.experimental import pallas as pl
from jax.experimental.pallas import tpu as pltpu
from jax.experimental.pallas import tpu_sc as plsc

NC = 2
NS = 16
NW = NC * NS

B = 16384
L = 50
D = 32
BL = 128
NBH = B // BL
BH_PER_W = NBH // NW
L_STRIDE = 4 * NBH * 1024
CH_STRIDE = NBH * 1024


def _sc_embedding(idx_b, weight, scale16):
    mesh = plsc.VectorSubcoreMesh(
        core_axis_name="c", subcore_axis_name="s",
        num_cores=NC, num_subcores=NS)

    @functools.partial(
        pl.kernel,
        out_type=jax.ShapeDtypeStruct((L * 4 * NBH, 8, BL), jnp.float32),
        mesh=mesh,
        compiler_params=pltpu.CompilerParams(
            use_tc_tiling_on_sc=False, needs_layout_passes=False),
        scratch_types=[
            pltpu.VMEM((BH_PER_W, L, BL), jnp.int32),
            pltpu.VMEM((BH_PER_W, BL, D), jnp.float32),
            pltpu.VMEM((BH_PER_W, D, 129), jnp.float32),
            pltpu.VMEM((16,), jnp.float32),
            pltpu.SemaphoreType.DMA,
            pltpu.SemaphoreType.DMA,
        ],
    )
    def k(idx_hbm, w_hbm, s_hbm, out_hbm, idx_v, rows_v, arr_v, s_v,
          sem_g, sem_w):
        cid = lax.axis_index("c")
        sid = lax.axis_index("s")
        wid = sid * NC + cid
        bh0 = wid * BH_PER_W

        pltpu.sync_copy(s_hbm, s_v)
        sf = jnp.exp(s_v[...])
        pltpu.sync_copy(idx_hbm.at[pl.ds(bh0, BH_PER_W)], idx_v)

        lane = lax.iota(jnp.int32, 16)
        base0 = lane * 129
        base1 = base0 + 16 * 129

        def issue_gather(l, j):
            pltpu.async_copy(w_hbm.at[idx_v.at[j, l]], rows_v.at[j], sem_g)

        def block(l, j, first, last):
            pltpu.make_async_copy(
                w_hbm.at[idx_v.at[j, l]], rows_v.at[j], sem_g).wait()
            if not first:
                for ch in range(4):
                    pltpu.make_async_copy(
                        arr_v.at[j, pl.ds(ch * 8, 8), pl.ds(0, BL)],
                        out_hbm.at[ch * NBH],
                        sem_w).wait()

            def rearr(bl, c2):
                v0 = rows_v[j, bl, pl.ds(0, 16)] * sf
                v1 = rows_v[j, bl, pl.ds(16, 16)] * sf
                blv = jnp.full((16,), bl, jnp.int32)
                plsc.store_scatter(arr_v.at[j], [lane, blv], v0)
                plsc.store_scatter(arr_v.at[j], [lane + 16, blv], v1)
                return c2

            lax.fori_loop(0, BL, rearr, 0, unroll=8)

            blk = l * (4 * NBH) + (bh0 + j)
            for ch in range(4):
                pltpu.async_copy(
                    arr_v.at[j, pl.ds(ch * 8, 8), pl.ds(0, BL)],
                    out_hbm.at[blk + ch * NBH],
                    sem_w)
            if not last:
                issue_gather(l + 1, j)

        for j in range(BH_PER_W):
            issue_gather(0, j)
        for j in range(BH_PER_W):
            block(0, j, first=True, last=False)

        def body(l, carry):
            for j in range(BH_PER_W):
                block(l, j, first=False, last=False)
            return carry

        lax.fori_loop(1, L - 1, body, 0)

        for j in range(BH_PER_W):
            block(L - 1, j, first=False, last=True)
        for j in range(BH_PER_W):
            for ch in range(4):
                pltpu.make_async_copy(
                    arr_v.at[j, pl.ds(ch * 8, 8), pl.ds(0, BL)],
                    out_hbm.at[ch * NBH],
                    sem_w).wait()

    return k(idx_b, weight, scale16)


def kernel(input, weight, scale):
    idx_b = input.astype(jnp.int32).reshape(NBH, BL, L).transpose(0, 2, 1)
    scale16 = jnp.broadcast_to(scale.astype(jnp.float32), (16,))
    flat = _sc_embedding(idx_b, weight, scale16)
    out5 = flat.reshape(L, 4, NBH, 8, BL)
    return out5.transpose(2, 4, 0, 1, 3).reshape(B, L, D)


# speedup vs baseline: 2.8039x; 1.0575x over previous
import functools

import jax
import jax.numpy as jnp
from jax import lax
from jax.experimental import pallas as pl
from jax.experimental.pallas import tpu as pltpu
from jax.experimental.pallas import tpu_sc as plsc

NC = 2
NS = 16
NW = NC * NS

B = 16384
L = 50
D = 32
BL = 128
NBH = B // BL
BH_PER_W = NBH // NW
L_STRIDE = 4 * NBH * 1024
CH_STRIDE = NBH * 1024


NBT = 7813
NBT_FULL = 7812
ABLOCKS = 244


def _sc_detile(wt):
    mesh = plsc.VectorSubcoreMesh(
        core_axis_name="c", subcore_axis_name="s",
        num_cores=NC, num_subcores=NS)

    @functools.partial(
        pl.kernel,
        out_type=jax.ShapeDtypeStruct((NBT * 32, BL), jnp.float32),
        mesh=mesh,
        compiler_params=pltpu.CompilerParams(
            use_tc_tiling_on_sc=True, needs_layout_passes=False),
        scratch_types=(
            [pltpu.VMEM((8, BL), jnp.float32) for _ in range(8)]
            + [pltpu.VMEM((128 * 33,), jnp.float32) for _ in range(2)]
            + [pltpu.VMEM((32, BL), jnp.float32) for _ in range(2)]
            + [pltpu.SemaphoreType.DMA, pltpu.SemaphoreType.DMA]
        ),
    )
    def ka(wt_hbm, tail_hbm, out_hbm,
           s00, s01, s02, s03, s10, s11, s12, s13,
           rows0, rows1, buf0, buf1, sem_a, sem_o):
        slabs = [[s00, s01, s02, s03], [s10, s11, s12, s13]]
        rows = [rows0, rows1]
        bufs = [buf0, buf1]
        cid = lax.axis_index("c")
        sid = lax.axis_index("s")
        wid = sid * NC + cid
        base = wid * ABLOCKS
        lane = lax.iota(jnp.int32, 16)

        def issue_slab(bt, d):
            for ct in range(4):
                pltpu.async_copy(
                    wt_hbm.at[pl.ds(ct * 8, 8), pl.ds(bt * BL, BL)],
                    slabs[d][ct], sem_a)

        def rearrange(d):
            def rg(g, c2):
                tg = (lane + g * 16) * 33
                for ct in range(4):
                    for cs in range(8):
                        c = ct * 8 + cs
                        v = slabs[d][ct][cs, pl.ds(g * 16, 16)]
                        plsc.store_scatter(rows[d], [tg + c], v)
                return c2
            lax.fori_loop(0, 8, rg, 0)

            def cp(i, c2):
                w0 = i * 16
                bsrc = w0 + w0 // 32
                r = w0 // 128
                y0 = w0 % 128
                bufs[d][r, pl.ds(y0, 16)] = rows[d][pl.ds(bsrc, 16)]
                return c2
            lax.fori_loop(0, 256, cp, 0, unroll=4)

        def ablock(bt, d, first, last):
            for ct in range(4):
                pltpu.make_async_copy(
                    wt_hbm.at[pl.ds(ct * 8, 8), pl.ds(bt * BL, BL)],
                    slabs[d][ct], sem_a).wait()
            if not first:
                pltpu.make_async_copy(
                    bufs[d], out_hbm.at[pl.ds(bt * 32, 32)], sem_o).wait()
            rearrange(d)
            pltpu.async_copy(bufs[d], out_hbm.at[pl.ds(bt * 32, 32)], sem_o)
            if not last:
                issue_slab(bt + 2, d)

        issue_slab(base, 0)
        issue_slab(base + 1, 1)
        ablock(base, 0, True, False)
        ablock(base + 1, 1, True, False)

        def body(i, carry):
            ablock(base + 2 * i, 0, False, False)
            ablock(base + 2 * i + 1, 1, False, False)
            return carry

        lax.fori_loop(1, ABLOCKS // 2 - 1, body, 0)
        ablock(base + ABLOCKS - 2, 0, False, True)
        ablock(base + ABLOCKS - 1, 1, False, True)
        for d in range(2):
            pltpu.make_async_copy(
                bufs[d], out_hbm.at[pl.ds(base * 32, 32)], sem_o).wait()

        for e in range(4):
            @pl.when(wid == e)
            def _():
                bt = 32 * ABLOCKS + e
                for ct in range(4):
                    pltpu.sync_copy(
                        wt_hbm.at[pl.ds(ct * 8, 8), pl.ds(bt * BL, BL)],
                        slabs[0][ct])
                rearrange(0)
                pltpu.sync_copy(bufs[0], out_hbm.at[pl.ds(bt * 32, 32)])

        @pl.when(wid == 5)
        def _():
            pltpu.sync_copy(tail_hbm, bufs[0].at[pl.ds(0, 16)])
            pltpu.sync_copy(bufs[0].at[pl.ds(0, 16)],
                            out_hbm.at[pl.ds(NBT_FULL * 32, 16)])

    tail = wt.T[NBT_FULL * BL:].reshape(16, BL)
    return ka(wt, tail)


def _sc_embedding(idx_b, weight, scale16):
    mesh = plsc.VectorSubcoreMesh(
        core_axis_name="c", subcore_axis_name="s",
        num_cores=NC, num_subcores=NS)

    @functools.partial(
        pl.kernel,
        out_type=jax.ShapeDtypeStruct((L * 4 * NBH, 8, BL), jnp.float32),
        mesh=mesh,
        compiler_params=pltpu.CompilerParams(
            use_tc_tiling_on_sc=False, needs_layout_passes=False),
        scratch_types=[
            pltpu.VMEM((BH_PER_W, L, BL), jnp.int32),
            pltpu.VMEM((BH_PER_W, BL, D), jnp.float32),
            pltpu.VMEM((BH_PER_W, D, 129), jnp.float32),
            pltpu.VMEM((16,), jnp.float32),
            pltpu.SemaphoreType.DMA,
            pltpu.SemaphoreType.DMA,
        ],
    )
    def k(idx_hbm, w_hbm, s_hbm, out_hbm, idx_v, rows_v, arr_v, s_v,
          sem_g, sem_w):
        cid = lax.axis_index("c")
        sid = lax.axis_index("s")
        wid = sid * NC + cid
        bh0 = wid * BH_PER_W

        pltpu.sync_copy(s_hbm, s_v)
        sf = jnp.exp(s_v[...])
        pltpu.sync_copy(idx_hbm.at[pl.ds(bh0, BH_PER_W)], idx_v)

        lane = lax.iota(jnp.int32, 16)
        base0 = lane * 129
        base1 = base0 + 16 * 129

        def issue_gather(l, j):
            pltpu.async_copy(w_hbm.at[idx_v.at[j, l]], rows_v.at[j], sem_g)

        def block(l, j, first, last):
            pltpu.make_async_copy(
                w_hbm.at[idx_v.at[j, l]], rows_v.at[j], sem_g).wait()
            if not first:
                for ch in range(4):
                    pltpu.make_async_copy(
                        arr_v.at[j, pl.ds(ch * 8, 8), pl.ds(0, BL)],
                        out_hbm.at[ch * NBH],
                        sem_w).wait()

            def rearr(bl, c2):
                v0 = rows_v[j, bl, pl.ds(0, 16)] * sf
                v1 = rows_v[j, bl, pl.ds(16, 16)] * sf
                blv = jnp.full((16,), bl, jnp.int32)
                plsc.store_scatter(arr_v.at[j], [lane, blv], v0)
                plsc.store_scatter(arr_v.at[j], [lane + 16, blv], v1)
                return c2

            lax.fori_loop(0, BL, rearr, 0, unroll=8)

            blk = l * (4 * NBH) + (bh0 + j)
            for ch in range(4):
                pltpu.async_copy(
                    arr_v.at[j, pl.ds(ch * 8, 8), pl.ds(0, BL)],
                    out_hbm.at[blk + ch * NBH],
                    sem_w)
            if not last:
                issue_gather(l + 1, j)

        for j in range(BH_PER_W):
            issue_gather(0, j)
        for j in range(BH_PER_W):
            block(0, j, first=True, last=False)

        def body(l, carry):
            for j in range(BH_PER_W):
                block(l, j, first=False, last=False)
            return carry

        lax.fori_loop(1, L - 1, body, 0)

        for j in range(BH_PER_W):
            block(L - 1, j, first=False, last=True)
        for j in range(BH_PER_W):
            for ch in range(4):
                pltpu.make_async_copy(
                    arr_v.at[j, pl.ds(ch * 8, 8), pl.ds(0, BL)],
                    out_hbm.at[ch * NBH],
                    sem_w).wait()

    return k(idx_b, weight, scale16)


def kernel(input, weight, scale):
    idx_b = input.astype(jnp.int32).reshape(NBH, BL, L).transpose(0, 2, 1)
    scale16 = jnp.broadcast_to(scale.astype(jnp.float32), (16,))
    w_lin = _sc_detile(weight.T).reshape(NBT * BL, D)

    flat = _sc_embedding(idx_b, w_lin, scale16)
    out5 = flat.reshape(L, 4, NBH, 8, BL)
    return out5.transpose(2, 4, 0, 1, 3).reshape(B, L, D)
